# baseline (speedup 1.0000x reference)
import jax
import jax.numpy as jnp
from jax import lax
from jax.experimental import pallas as pl
from jax.experimental.pallas import tpu as pltpu
from jax.experimental.pallas import tpu_sc as plsc

N_EMB = 16384
D = 512
DP = D // 2
N_TRIPLETS = 200000
MARGIN = 1.0

NW = 32
T_PER_W = 6272
T_PAD = NW * T_PER_W
CHUNK = 32
N_CHUNKS = T_PER_W // CHUNK


def _tl_body(emb_hbm, t0_hbm, t1_hbm, t2_hbm, out_hbm,
             t0_v, t1_v, t2_v,
             a0_buf, p0_buf, n0_buf, a1_buf, p1_buf, n1_buf,
             loss_v, sem0, sem1):
    cid = lax.axis_index("c")
    sid = lax.axis_index("s")
    wid = sid * 2 + cid
    base = wid * T_PER_W

    pltpu.sync_copy(t0_hbm.at[pl.ds(base, T_PER_W)], t0_v)
    pltpu.sync_copy(t1_hbm.at[pl.ds(base, T_PER_W)], t1_v)
    pltpu.sync_copy(t2_hbm.at[pl.ds(base, T_PER_W)], t2_v)

    lanes = lax.iota(jnp.int32, 16)
    lanes_hi = lanes + 16
    zero16 = jnp.zeros((16,), jnp.float32)

    def issue(g, a_b, p_b, n_b, sem):
        off = g * CHUNK
        pltpu.async_copy(emb_hbm.at[t0_v.at[pl.ds(off, CHUNK)]], a_b, sem)
        pltpu.async_copy(emb_hbm.at[t1_v.at[pl.ds(off, CHUNK)]], p_b, sem)
        pltpu.async_copy(emb_hbm.at[t2_v.at[pl.ds(off, CHUNK)]], n_b, sem)

    def drain(a_b, p_b, n_b, sem):
        pltpu.make_async_copy(emb_hbm.at[t0_v.at[pl.ds(0, CHUNK)]], a_b, sem).wait()
        pltpu.make_async_copy(emb_hbm.at[t1_v.at[pl.ds(0, CHUNK)]], p_b, sem).wait()
        pltpu.make_async_copy(emb_hbm.at[t2_v.at[pl.ds(0, CHUNK)]], n_b, sem).wait()

    def compute(a_b, p_b, n_b, loss_acc):
        def pair_terms(wa, wp, wn):
            ba = plsc.bitcast(wa, jnp.bfloat16)
            bp = plsc.bitcast(wp, jnp.bfloat16)
            bn = plsc.bitcast(wn, jnp.bfloat16)
            alo, ahi = plsc.unpack(ba, format=plsc.PackFormat.INTERLEAVED,
                                   preferred_element_type=jnp.float32)
            plo, phi = plsc.unpack(bp, format=plsc.PackFormat.INTERLEAVED,
                                   preferred_element_type=jnp.float32)
            nlo, nhi = plsc.unpack(bn, format=plsc.PackFormat.INTERLEAVED,
                                   preferred_element_type=jnp.float32)
            dlo = plo - nlo
            dhi = phi - nhi
            slo = (plo + nlo) - (alo + alo)
            shi = (phi + nhi) - (ahi + ahi)
            return dlo * slo + dhi * shi

        def d_body(c, carry):
            acc0, acc1 = carry
            col = (lanes + c) & (DP - 1)
            a0 = plsc.load_gather(a_b, [lanes, col])
            p0 = plsc.load_gather(p_b, [lanes, col])
            n0 = plsc.load_gather(n_b, [lanes, col])
            a1 = plsc.load_gather(a_b, [lanes_hi, col])
            p1 = plsc.load_gather(p_b, [lanes_hi, col])
            n1 = plsc.load_gather(n_b, [lanes_hi, col])
            return (acc0 + pair_terms(a0, p0, n0),
                    acc1 + pair_terms(a1, p1, n1))

        acc0, acc1 = lax.fori_loop(
            0, DP, d_body, (zero16, zero16), unroll=4)
        l0 = jnp.maximum(acc0 + MARGIN, 0.0)
        l1 = jnp.maximum(acc1 + MARGIN, 0.0)
        return loss_acc + l0 + l1

    issue(0, a0_buf, p0_buf, n0_buf, sem0)

    def pair_body(h, loss_acc):
        g0 = 2 * h
        issue(g0 + 1, a1_buf, p1_buf, n1_buf, sem1)
        drain(a0_buf, p0_buf, n0_buf, sem0)
        loss_acc = compute(a0_buf, p0_buf, n0_buf, loss_acc)

        @pl.when(g0 + 2 < N_CHUNKS)
        def _():
            issue(g0 + 2, a0_buf, p0_buf, n0_buf, sem0)

        drain(a1_buf, p1_buf, n1_buf, sem1)
        return compute(a1_buf, p1_buf, n1_buf, loss_acc)

    loss_acc = lax.fori_loop(0, N_CHUNKS // 2, pair_body, zero16)
    loss_v[...] = loss_acc
    pltpu.sync_copy(loss_v, out_hbm.at[wid])


@jax.jit
def _tl_call(embeddings, t0, t1, t2):
    mesh = plsc.VectorSubcoreMesh(core_axis_name="c", subcore_axis_name="s")
    row = pltpu.VMEM((CHUNK, DP), jnp.int32)
    fn = pl.kernel(
        _tl_body,
        out_type=jax.ShapeDtypeStruct((NW, 16), jnp.float32),
        mesh=mesh,
        scratch_types=[
            pltpu.VMEM((T_PER_W,), jnp.int32),
            pltpu.VMEM((T_PER_W,), jnp.int32),
            pltpu.VMEM((T_PER_W,), jnp.int32),
            row, row, row, row, row, row,
            pltpu.VMEM((16,), jnp.float32),
            pltpu.SemaphoreType.DMA,
            pltpu.SemaphoreType.DMA,
        ],
        compiler_params=pltpu.CompilerParams(
            use_tc_tiling_on_sc=False, needs_layout_passes=False),
    )
    return fn(embeddings, t0, t1, t2)


def kernel(embeddings, target, triplets):
    del target
    emb_packed = jax.lax.bitcast_convert_type(
        embeddings.astype(jnp.bfloat16).reshape(N_EMB, DP, 2), jnp.int32)
    tpad = jnp.pad(triplets, ((0, T_PAD - N_TRIPLETS), (0, 0)))
    t0 = tpad[:, 0]
    t1 = tpad[:, 1]
    t2 = tpad[:, 2]
    partials = _tl_call(emb_packed, t0, t1, t2)
    total = jnp.sum(partials) - jnp.float32(T_PAD - N_TRIPLETS)
    return (total / N_TRIPLETS, triplets.shape[0])


# speedup vs baseline: 1.1833x; 1.1833x over previous
import jax
import jax.numpy as jnp
from jax import lax
from jax.experimental import pallas as pl
from jax.experimental.pallas import tpu as pltpu
from jax.experimental.pallas import tpu_sc as plsc

N_EMB = 16384
D = 512
DP = D // 2
N_TRIPLETS = 200000
MARGIN = 1.0

NW = 32
T_PER_W = 6272
T_PAD = NW * T_PER_W
CHUNK = 32
N_CHUNKS = T_PER_W // CHUNK


def _tl_body(emb_hbm, t0_hbm, t1_hbm, t2_hbm, out_hbm,
             t0_v, t1_v, t2_v,
             a0_buf, p0_buf, n0_buf, a1_buf, p1_buf, n1_buf,
             loss_v, sem0, sem1):
    cid = lax.axis_index("c")
    sid = lax.axis_index("s")
    wid = sid * 2 + cid
    base = wid * T_PER_W

    pltpu.sync_copy(t0_hbm.at[pl.ds(base, T_PER_W)], t0_v)
    pltpu.sync_copy(t1_hbm.at[pl.ds(base, T_PER_W)], t1_v)
    pltpu.sync_copy(t2_hbm.at[pl.ds(base, T_PER_W)], t2_v)

    lanes = lax.iota(jnp.int32, 16)
    lanes_hi = lanes + 16
    zero16 = jnp.zeros((16,), jnp.float32)

    def issue(g, a_b, p_b, n_b, sem):
        off = g * CHUNK
        pltpu.async_copy(emb_hbm.at[t0_v.at[pl.ds(off, CHUNK)]], a_b, sem)
        pltpu.async_copy(emb_hbm.at[t1_v.at[pl.ds(off, CHUNK)]], p_b, sem)
        pltpu.async_copy(emb_hbm.at[t2_v.at[pl.ds(off, CHUNK)]], n_b, sem)

    def drain(a_b, p_b, n_b, sem):
        pltpu.make_async_copy(emb_hbm.at[t0_v.at[pl.ds(0, CHUNK)]], a_b, sem).wait()
        pltpu.make_async_copy(emb_hbm.at[t1_v.at[pl.ds(0, CHUNK)]], p_b, sem).wait()
        pltpu.make_async_copy(emb_hbm.at[t2_v.at[pl.ds(0, CHUNK)]], n_b, sem).wait()

    def compute(a_b, p_b, n_b, loss_acc):
        def pair_terms(wa, wp, wn):
            ba = plsc.bitcast(wa, jnp.bfloat16)
            bp = plsc.bitcast(wp, jnp.bfloat16)
            bn = plsc.bitcast(wn, jnp.bfloat16)
            d = bp - bn
            f = (bp + bn) - (ba + ba)
            prod = d * f
            return plsc.unpack(prod, format=plsc.PackFormat.INTERLEAVED,
                               preferred_element_type=jnp.float32)

        def d_body(c, carry):
            acc0, acc1, acc2, acc3 = carry
            col = (lanes + c) & (DP - 1)
            a0 = plsc.load_gather(a_b, [lanes, col])
            p0 = plsc.load_gather(p_b, [lanes, col])
            n0 = plsc.load_gather(n_b, [lanes, col])
            a1 = plsc.load_gather(a_b, [lanes_hi, col])
            p1 = plsc.load_gather(p_b, [lanes_hi, col])
            n1 = plsc.load_gather(n_b, [lanes_hi, col])
            lo0, hi0 = pair_terms(a0, p0, n0)
            lo1, hi1 = pair_terms(a1, p1, n1)
            return (acc0 + lo0, acc1 + hi0, acc2 + lo1, acc3 + hi1)

        acc0, acc1, acc2, acc3 = lax.fori_loop(
            0, DP, d_body, (zero16, zero16, zero16, zero16), unroll=4)
        l0 = jnp.maximum((acc0 + acc1) + MARGIN, 0.0)
        l1 = jnp.maximum((acc2 + acc3) + MARGIN, 0.0)
        return loss_acc + l0 + l1

    issue(0, a0_buf, p0_buf, n0_buf, sem0)

    def pair_body(h, loss_acc):
        g0 = 2 * h
        issue(g0 + 1, a1_buf, p1_buf, n1_buf, sem1)
        drain(a0_buf, p0_buf, n0_buf, sem0)
        loss_acc = compute(a0_buf, p0_buf, n0_buf, loss_acc)

        @pl.when(g0 + 2 < N_CHUNKS)
        def _():
            issue(g0 + 2, a0_buf, p0_buf, n0_buf, sem0)

        drain(a1_buf, p1_buf, n1_buf, sem1)
        return compute(a1_buf, p1_buf, n1_buf, loss_acc)

    loss_acc = lax.fori_loop(0, N_CHUNKS // 2, pair_body, zero16)
    loss_v[...] = loss_acc
    pltpu.sync_copy(loss_v, out_hbm.at[wid])


@jax.jit
def _tl_call(embeddings, t0, t1, t2):
    mesh = plsc.VectorSubcoreMesh(core_axis_name="c", subcore_axis_name="s")
    row = pltpu.VMEM((CHUNK, DP), jnp.int32)
    fn = pl.kernel(
        _tl_body,
        out_type=jax.ShapeDtypeStruct((NW, 16), jnp.float32),
        mesh=mesh,
        scratch_types=[
            pltpu.VMEM((T_PER_W,), jnp.int32),
            pltpu.VMEM((T_PER_W,), jnp.int32),
            pltpu.VMEM((T_PER_W,), jnp.int32),
            row, row, row, row, row, row,
            pltpu.VMEM((16,), jnp.float32),
            pltpu.SemaphoreType.DMA,
            pltpu.SemaphoreType.DMA,
        ],
        compiler_params=pltpu.CompilerParams(
            use_tc_tiling_on_sc=False, needs_layout_passes=False),
    )
    return fn(embeddings, t0, t1, t2)


def kernel(embeddings, target, triplets):
    del target
    emb_packed = jax.lax.bitcast_convert_type(
        embeddings.astype(jnp.bfloat16).reshape(N_EMB, DP, 2), jnp.int32)
    tpad = jnp.pad(triplets, ((0, T_PAD - N_TRIPLETS), (0, 0)))
    t0 = tpad[:, 0]
    t1 = tpad[:, 1]
    t2 = tpad[:, 2]
    partials = _tl_call(emb_packed, t0, t1, t2)
    total = jnp.sum(partials) - jnp.float32(T_PAD - N_TRIPLETS)
    return (total / N_TRIPLETS, triplets.shape[0])
